# Initial kernel scaffold; baseline (speedup 1.0000x reference)
import functools
import jax
import jax.numpy as jnp
from jax import lax
from jax.experimental import pallas as pl
from jax.experimental.pallas import tpu as pltpu
from jax.experimental.pallas import tpu_sc as plsc

N_NODES = 10000
NP = 10240
E_TOT = 320000
D = 128
ED = 16
NC, NS = 2, 16
NW = NC * NS
EPT = E_TOT // NW
NPT = NP // NW
NPS = NP // NS
CPT = D // NW

CH_A = 2000
CH_E = 2000
CH_G = 400

_mesh = functools.partial(
    plsc.VectorSubcoreMesh, core_axis_name="c", subcore_axis_name="s")

_SHIFT_DNUMS = lax.GatherDimensionNumbers(
    offset_dims=(), collapsed_slice_dims=(0,), start_index_map=(0,))


def _shift_up(v):
    idx = jnp.minimum(lax.iota(jnp.int32, 16) + 1, 15).reshape(16, 1)
    return lax.gather(v, idx, _SHIFT_DNUMS, (1,),
                      mode=lax.GatherScatterMode.PROMISE_IN_BOUNDS)


@functools.partial(
    pl.kernel,
    out_type=jax.ShapeDtypeStruct((2 * NC * NP,), jnp.int32),
    mesh=_mesh(),
    scratch_types=[
        pltpu.VMEM((CH_A,), jnp.int32),
        pltpu.VMEM((CH_A,), jnp.int32),
        pltpu.VMEM((NP,), jnp.int32),
        pltpu.VMEM((NP,), jnp.int32),
        pltpu.VMEM((NPS,), jnp.int32),
        pltpu.VMEM((NPS,), jnp.int32),
        pltpu.VMEM_SHARED((NS, NP), jnp.int32),
        pltpu.VMEM_SHARED((NS, NP), jnp.int32),
    ],
)
def _winners_kernel(src_hbm, dst_hbm, wout, sbuf, dbuf, wsrc, wdst,
                    acc_s, acc_d, sh_src, sh_dst):
    cid = lax.axis_index("c")
    sid = lax.axis_index("s")
    wid = cid * NS + sid
    neg1 = jnp.full((16,), -1, jnp.int32)
    lane = lax.iota(jnp.int32, 16)

    def init_body(i, _):
        wsrc[pl.ds(i * 16, 16)] = neg1
        wdst[pl.ds(i * 16, 16)] = neg1
        return 0
    lax.fori_loop(0, NP // 16, init_body, 0)

    for c in range(EPT // CH_A):
        base = wid * EPT + c * CH_A
        pltpu.sync_copy(src_hbm.at[pl.ds(base, CH_A)], sbuf)
        pltpu.sync_copy(dst_hbm.at[pl.ds(base, CH_A)], dbuf)

        def chunk_body(k, _):
            e0 = base + k * 16
            for buf, tbl in ((sbuf, wsrc), (dbuf, wdst)):
                idxv = buf[pl.ds(k * 16, 16)]
                key = idxv * 16 + lane
                ks, _unused = plsc.sort_key_val(key, key)
                node = lax.shift_right_logical(ks, 4)
                nxt = lax.shift_right_logical(_shift_up(ks), 4)
                win = (lane == 15) | (node != nxt)
                e_val = e0 + (ks & 15)
                plsc.store_scatter(tbl, [node], e_val, mask=win)
            return 0
        lax.fori_loop(0, CH_A // 16, chunk_body, 0)

    pltpu.sync_copy(wsrc, sh_src.at[sid])
    pltpu.sync_copy(wdst, sh_dst.at[sid])
    plsc.subcore_barrier()

    def init2(i, _):
        acc_s[pl.ds(i * 16, 16)] = neg1
        acc_d[pl.ds(i * 16, 16)] = neg1
        return 0
    lax.fori_loop(0, NPS // 16, init2, 0)

    for j in range(NS):
        pltpu.sync_copy(sh_src.at[j], wsrc)
        pltpu.sync_copy(sh_dst.at[j], wdst)

        def merge_body(k, _):
            off = sid * NPS + k * 16
            acc_s[pl.ds(k * 16, 16)] = jnp.maximum(
                acc_s[pl.ds(k * 16, 16)], wsrc[pl.ds(off, 16)])
            acc_d[pl.ds(k * 16, 16)] = jnp.maximum(
                acc_d[pl.ds(k * 16, 16)], wdst[pl.ds(off, 16)])
            return 0
        lax.fori_loop(0, NPS // 16, merge_body, 0)

    pltpu.sync_copy(acc_s, wout.at[pl.ds((cid * 2 + 0) * NP + sid * NPS, NPS)])
    pltpu.sync_copy(acc_d, wout.at[pl.ds((cid * 2 + 1) * NP + sid * NPS, NPS)])


@functools.partial(
    pl.kernel,
    out_type=[jax.ShapeDtypeStruct((NP, D), jnp.float32),
              jax.ShapeDtypeStruct((NP, D), jnp.float32)],
    mesh=_mesh(),
    scratch_types=[
        pltpu.VMEM((NPT,), jnp.int32),
        pltpu.VMEM((NPT,), jnp.int32),
        pltpu.VMEM((NPT,), jnp.int32),
        pltpu.VMEM((NPT,), jnp.int32),
        pltpu.VMEM((NPT, D), jnp.float32),
        pltpu.SemaphoreType.DMA,
    ],
)
def _gather_nodes_kernel(wm_hbm, xsrc_hbm, xdst_hbm, xs_out, xd_out,
                         w0, w1, wm, gidx, rows, sem):
    cid = lax.axis_index("c")
    sid = lax.axis_index("s")
    wid = cid * NS + sid
    nb = wid * NPT
    lane = lax.iota(jnp.int32, 16)
    zero16 = jnp.zeros((16,), jnp.float32)

    for tsel, xtab, outref in ((0, xsrc_hbm, xs_out), (1, xdst_hbm, xd_out)):
        pltpu.sync_copy(wm_hbm.at[pl.ds(tsel * NP + nb, NPT)], w0)
        pltpu.sync_copy(wm_hbm.at[pl.ds((2 + tsel) * NP + nb, NPT)], w1)

        def merge_body(k, _):
            m = jnp.maximum(w0[pl.ds(k * 16, 16)], w1[pl.ds(k * 16, 16)])
            wm[pl.ds(k * 16, 16)] = m
            gidx[pl.ds(k * 16, 16)] = jnp.maximum(m, 0)
            return 0
        lax.fori_loop(0, NPT // 16, merge_body, 0)

        pltpu.async_copy(xtab.at[gidx], rows, sem).wait()

        def mask_body(k, _):
            mv = wm[pl.ds(k * 16, 16)] < 0
            rid = k * 16 + lane
            for col in range(D):
                plsc.store_scatter(
                    rows, [rid, jnp.full((16,), col, jnp.int32)],
                    zero16, mask=mv)
            return 0
        lax.fori_loop(0, NPT // 16, mask_body, 0)

        pltpu.sync_copy(rows, outref.at[pl.ds(nb, NPT)])


def _proj_body(xs_ref, xd_ref, ws_ref, wd_ref, wmx_ref, wself_ref, bsd_ref,
               pt_ref, st_ref):
    xproj = lax.dot_general(xs_ref[...], ws_ref[...], (((1,), (1,)), ((), ())),
                            preferred_element_type=jnp.float32)
    xproj += lax.dot_general(xd_ref[...], wd_ref[...], (((1,), (1,)), ((), ())),
                             preferred_element_type=jnp.float32)
    xproj += bsd_ref[...]
    pt_ref[...] = lax.dot_general(wmx_ref[...], xproj, (((0,), (1,)), ((), ())),
                                  preferred_element_type=jnp.float32)
    st_ref[...] = lax.dot_general(wself_ref[...], xproj, (((0,), (1,)), ((), ())),
                                  preferred_element_type=jnp.float32)


def _proj_call(xs, xd, w_src, w_dst, w_msg_x, w_self, b_sd):
    bn = 2048
    return pl.pallas_call(
        _proj_body,
        grid=(NP // bn,),
        in_specs=[
            pl.BlockSpec((bn, D), lambda i: (i, 0)),
            pl.BlockSpec((bn, D), lambda i: (i, 0)),
            pl.BlockSpec((D, D), lambda i: (0, 0)),
            pl.BlockSpec((D, D), lambda i: (0, 0)),
            pl.BlockSpec((D, D), lambda i: (0, 0)),
            pl.BlockSpec((D, D), lambda i: (0, 0)),
            pl.BlockSpec((1, D), lambda i: (0, 0)),
        ],
        out_specs=[
            pl.BlockSpec((D, bn), lambda i: (0, i)),
            pl.BlockSpec((D, bn), lambda i: (0, i)),
        ],
        out_shape=[jax.ShapeDtypeStruct((D, NP), jnp.float32),
                   jax.ShapeDtypeStruct((D, NP), jnp.float32)],
    )(xs, xd, w_src, w_dst, w_msg_x, w_self, b_sd)


def _qt_body(msg_ref, we_ref, qt_ref):
    qt_ref[...] = lax.dot_general(
        we_ref[...], msg_ref[...], (((0,), (1,)), ((), ())),
        preferred_element_type=jnp.float32)


def _qt_call(msg, w_e):
    be = 2000
    return pl.pallas_call(
        _qt_body,
        grid=(E_TOT // be,),
        in_specs=[
            pl.BlockSpec((be, ED), lambda i: (i, 0)),
            pl.BlockSpec((ED, D), lambda i: (0, 0)),
        ],
        out_specs=pl.BlockSpec((D, be), lambda i: (0, i)),
        out_shape=jax.ShapeDtypeStruct((D, E_TOT), jnp.float32),
    )(msg, w_e)


@functools.partial(
    pl.kernel,
    out_type=jax.ShapeDtypeStruct((D * NP,), jnp.float32),
    mesh=_mesh(),
    scratch_types=[
        pltpu.VMEM((CH_E,), jnp.int32),
        pltpu.VMEM((CH_E,), jnp.int32),
        pltpu.VMEM((CPT, CH_E), jnp.float32),
        pltpu.VMEM((CPT, NP), jnp.float32),
        pltpu.VMEM((CPT, NP), jnp.float32),
    ],
)
def _edge_kernel(pt_hbm, qt_hbm, src_hbm, dst_hbm, agg_out,
                 sbuf, dbuf, qbuf, pstrip, agg):
    cid = lax.axis_index("c")
    sid = lax.axis_index("s")
    wid = cid * NS + sid
    zero16 = jnp.zeros((16,), jnp.float32)
    rcols = [jnp.full((16,), r, jnp.int32) for r in range(CPT)]

    for r in range(CPT):
        pltpu.sync_copy(pt_hbm.at[pl.ds((CPT * wid + r) * NP, NP)],
                        pstrip.at[r])

    def zero_body(i, _):
        for r in range(CPT):
            agg[r, pl.ds(i * 16, 16)] = zero16
        return 0
    lax.fori_loop(0, NP // 16, zero_body, 0)

    def chunk_body(ci, _):
        base = ci * CH_E
        pltpu.sync_copy(src_hbm.at[pl.ds(base, CH_E)], sbuf)
        pltpu.sync_copy(dst_hbm.at[pl.ds(base, CH_E)], dbuf)
        for r in range(CPT):
            pltpu.sync_copy(
                qt_hbm.at[pl.ds((CPT * wid + r) * E_TOT + base, CH_E)],
                qbuf.at[r])

        def batch_body(k, _):
            sidx = sbuf[pl.ds(k * 16, 16)]
            didx = dbuf[pl.ds(k * 16, 16)]
            for r in range(CPT):
                pv = plsc.load_gather(pstrip, [rcols[r], sidx])
                qv = qbuf[r, pl.ds(k * 16, 16)]
                m = jnp.maximum(pv + qv, 0.0)
                plsc.addupdate_scatter(agg, [rcols[r], didx], m)
            return 0
        lax.fori_loop(0, CH_E // 16, batch_body, 0)
        return 0
    lax.fori_loop(0, E_TOT // CH_E, chunk_body, 0)

    for r in range(CPT):
        pltpu.sync_copy(agg.at[r],
                        agg_out.at[pl.ds((CPT * wid + r) * NP, NP)])


def _hfinal_body(aggt_ref, st_ref, benc_ref, eye_ref, h_ref):
    x = aggt_ref[...] + st_ref[...] + benc_ref[...]
    xt = lax.dot_general(x, eye_ref[...], (((0,), (0,)), ((), ())),
                         preferred_element_type=jnp.float32)
    h_ref[...] = jnp.maximum(xt, 0.0)


def _hfinal_call(aggt, st, b_enc2, eye):
    bn = 2048
    return pl.pallas_call(
        _hfinal_body,
        grid=(NP // bn,),
        in_specs=[
            pl.BlockSpec((D, bn), lambda i: (0, i)),
            pl.BlockSpec((D, bn), lambda i: (0, i)),
            pl.BlockSpec((D, 1), lambda i: (0, 0)),
            pl.BlockSpec((D, D), lambda i: (0, 0)),
        ],
        out_specs=pl.BlockSpec((bn, D), lambda i: (i, 0)),
        out_shape=jax.ShapeDtypeStruct((NP, D), jnp.float32),
    )(aggt, st, b_enc2, eye)


@functools.partial(
    pl.kernel,
    out_type=[jax.ShapeDtypeStruct((E_TOT, D), jnp.float32),
              jax.ShapeDtypeStruct((E_TOT, D), jnp.float32)],
    mesh=_mesh(),
    scratch_types=[
        pltpu.VMEM((CH_G,), jnp.int32),
        pltpu.VMEM((CH_G,), jnp.int32),
        pltpu.VMEM((CH_G, D), jnp.float32),
        pltpu.VMEM((CH_G, D), jnp.float32),
        pltpu.SemaphoreType.DMA,
        pltpu.SemaphoreType.DMA,
    ],
)
def _output_gather_kernel(h_hbm, src_hbm, dst_hbm, out_s, out_d,
                          sidx, didx, srows, drows, sem_s, sem_d):
    cid = lax.axis_index("c")
    sid = lax.axis_index("s")
    wid = cid * NS + sid

    def chunk_body(j, _):
        base = wid * EPT + j * CH_G
        pltpu.sync_copy(src_hbm.at[pl.ds(base, CH_G)], sidx)
        pltpu.sync_copy(dst_hbm.at[pl.ds(base, CH_G)], didx)
        cs = pltpu.async_copy(h_hbm.at[sidx], srows, sem_s)
        cd = pltpu.async_copy(h_hbm.at[didx], drows, sem_d)
        cs.wait()
        pltpu.sync_copy(srows, out_s.at[pl.ds(base, CH_G)])
        cd.wait()
        pltpu.sync_copy(drows, out_d.at[pl.ds(base, CH_G)])
        return 0
    lax.fori_loop(0, EPT // CH_G, chunk_body, 0)


def kernel(edge_index, t, msg, x_src, x_dst, W_src, b_src, W_dst, b_dst,
           W_self, W_msg, b_enc):
    del t
    src = edge_index[0].astype(jnp.int32)
    dst = edge_index[1].astype(jnp.int32)

    wmerged = _winners_kernel(src, dst)
    xs, xd = _gather_nodes_kernel(wmerged, x_src, x_dst)

    b_sd = (b_src + b_dst).reshape(1, D).astype(jnp.float32)
    pt, st = _proj_call(xs, xd, W_src, W_dst, W_msg[:D], W_self, b_sd)
    qt = _qt_call(msg, W_msg[D:])

    aggt_flat = _edge_kernel(pt.reshape(D * NP), qt.reshape(D * E_TOT),
                             src, dst)

    b_enc2 = b_enc.reshape(D, 1).astype(jnp.float32)
    eye = jnp.eye(D, dtype=jnp.float32)
    h = _hfinal_call(aggt_flat.reshape(D, NP), st, b_enc2, eye)

    h_src, h_dst = _output_gather_kernel(h, src, dst)
    return (h_src, h_dst)


# speedup vs baseline: 7.2958x; 7.2958x over previous
import functools
import jax
import jax.numpy as jnp
from jax import lax
from jax.experimental import pallas as pl
from jax.experimental.pallas import tpu as pltpu
from jax.experimental.pallas import tpu_sc as plsc

N_NODES = 10000
NP = 10240
E_TOT = 320000
D = 128
ED = 16
NC, NS = 2, 16
NW = NC * NS
EPT = E_TOT // NW
NPT = NP // NW
NPS = NP // NS
CPT = D // NW

CH_A = 2000
CH_E = 2000
CH_G = 400

_mesh = functools.partial(
    plsc.VectorSubcoreMesh, core_axis_name="c", subcore_axis_name="s")

_SC_PARAMS = pltpu.CompilerParams(needs_layout_passes=False)

@functools.partial(
    pl.kernel,
    out_type=jax.ShapeDtypeStruct((2 * NC * NP,), jnp.int32),
    mesh=_mesh(),
    compiler_params=_SC_PARAMS,
    scratch_types=[
        pltpu.VMEM((CH_A,), jnp.int32),
        pltpu.VMEM((CH_A,), jnp.int32),
        pltpu.VMEM((NP,), jnp.int32),
        pltpu.VMEM((NP,), jnp.int32),
        pltpu.VMEM((NPS,), jnp.int32),
        pltpu.VMEM((NPS,), jnp.int32),
        pltpu.VMEM_SHARED((NS * NP,), jnp.int32),
        pltpu.VMEM_SHARED((NS * NP,), jnp.int32),
    ],
)
def _winners_kernel(src_hbm, dst_hbm, wout, sbuf, dbuf, wsrc, wdst,
                    acc_s, acc_d, sh_src, sh_dst):
    cid = lax.axis_index("c")
    sid = lax.axis_index("s")
    wid = cid * NS + sid
    neg1 = jnp.full((16,), -1, jnp.int32)
    lane = lax.iota(jnp.int32, 16)
    lane_masks = [lane == i for i in range(16)]

    def init_body(i, _):
        wsrc[pl.ds(i * 16, 16)] = neg1
        wdst[pl.ds(i * 16, 16)] = neg1
        return 0
    lax.fori_loop(0, NP // 16, init_body, 0)

    for c in range(EPT // CH_A):
        base = wid * EPT + c * CH_A
        pltpu.sync_copy(src_hbm.at[pl.ds(base, CH_A)], sbuf)
        pltpu.sync_copy(dst_hbm.at[pl.ds(base, CH_A)], dbuf)

        def chunk_body(k, _):
            e0 = base + k * 16
            for buf, tbl in ((sbuf, wsrc), (dbuf, wdst)):
                idxv = buf[pl.ds(k * 16, 16)]
                e_val = e0 + lane
                for i in range(16):
                    plsc.store_scatter(tbl, [idxv], e_val, mask=lane_masks[i])
            return 0
        lax.fori_loop(0, CH_A // 16, chunk_body, 0)

    pltpu.sync_copy(wsrc, sh_src.at[pl.ds(sid * NP, NP)])
    pltpu.sync_copy(wdst, sh_dst.at[pl.ds(sid * NP, NP)])
    plsc.subcore_barrier()

    def init2(i, _):
        acc_s[pl.ds(i * 16, 16)] = neg1
        acc_d[pl.ds(i * 16, 16)] = neg1
        return 0
    lax.fori_loop(0, NPS // 16, init2, 0)

    for j in range(NS):
        pltpu.sync_copy(sh_src.at[pl.ds(j * NP, NP)], wsrc)
        pltpu.sync_copy(sh_dst.at[pl.ds(j * NP, NP)], wdst)

        def merge_body(k, _):
            off = sid * NPS + k * 16
            acc_s[pl.ds(k * 16, 16)] = jnp.maximum(
                acc_s[pl.ds(k * 16, 16)], wsrc[pl.ds(off, 16)])
            acc_d[pl.ds(k * 16, 16)] = jnp.maximum(
                acc_d[pl.ds(k * 16, 16)], wdst[pl.ds(off, 16)])
            return 0
        lax.fori_loop(0, NPS // 16, merge_body, 0)

    pltpu.sync_copy(acc_s, wout.at[pl.ds((cid * 2 + 0) * NP + sid * NPS, NPS)])
    pltpu.sync_copy(acc_d, wout.at[pl.ds((cid * 2 + 1) * NP + sid * NPS, NPS)])


@functools.partial(
    pl.kernel,
    out_type=[jax.ShapeDtypeStruct((NP, D), jnp.float32),
              jax.ShapeDtypeStruct((NP, D), jnp.float32)],
    mesh=_mesh(),
    compiler_params=_SC_PARAMS,
    scratch_types=[
        pltpu.VMEM((NPT,), jnp.int32),
        pltpu.VMEM((NPT,), jnp.int32),
        pltpu.VMEM((NPT,), jnp.int32),
        pltpu.VMEM((NPT,), jnp.int32),
        pltpu.VMEM((NPT, D), jnp.float32),
        pltpu.SemaphoreType.DMA,
    ],
)
def _gather_nodes_kernel(wm_hbm, xsrc_hbm, xdst_hbm, xs_out, xd_out,
                         w0, w1, wm, gidx, rows, sem):
    cid = lax.axis_index("c")
    sid = lax.axis_index("s")
    wid = cid * NS + sid
    nb = wid * NPT
    lane = lax.iota(jnp.int32, 16)
    zero16 = jnp.zeros((16,), jnp.float32)

    for tsel, xtab, outref in ((0, xsrc_hbm, xs_out), (1, xdst_hbm, xd_out)):
        pltpu.sync_copy(wm_hbm.at[pl.ds(tsel * NP + nb, NPT)], w0)
        pltpu.sync_copy(wm_hbm.at[pl.ds((2 + tsel) * NP + nb, NPT)], w1)

        def merge_body(k, _):
            m = jnp.maximum(w0[pl.ds(k * 16, 16)], w1[pl.ds(k * 16, 16)])
            wm[pl.ds(k * 16, 16)] = m
            gidx[pl.ds(k * 16, 16)] = jnp.maximum(m, 0)
            return 0
        lax.fori_loop(0, NPT // 16, merge_body, 0)

        pltpu.async_copy(xtab.at[gidx], rows, sem).wait()

        def mask_body(k, _):
            mv = wm[pl.ds(k * 16, 16)] < 0
            rid = k * 16 + lane
            for col in range(D):
                plsc.store_scatter(
                    rows, [rid, jnp.full((16,), col, jnp.int32)],
                    zero16, mask=mv)
            return 0
        lax.fori_loop(0, NPT // 16, mask_body, 0)

        pltpu.sync_copy(rows, outref.at[pl.ds(nb, NPT)])


def _proj_body(xs_ref, xd_ref, ws_ref, wd_ref, wmx_ref, wself_ref, bsd_ref,
               pt_ref, st_ref):
    xproj = lax.dot_general(xs_ref[...], ws_ref[...], (((1,), (1,)), ((), ())),
                            preferred_element_type=jnp.float32)
    xproj += lax.dot_general(xd_ref[...], wd_ref[...], (((1,), (1,)), ((), ())),
                             preferred_element_type=jnp.float32)
    xproj += bsd_ref[...]
    pt_ref[...] = lax.dot_general(wmx_ref[...], xproj, (((0,), (1,)), ((), ())),
                                  preferred_element_type=jnp.float32)
    st_ref[...] = lax.dot_general(wself_ref[...], xproj, (((0,), (1,)), ((), ())),
                                  preferred_element_type=jnp.float32)


def _proj_call(xs, xd, w_src, w_dst, w_msg_x, w_self, b_sd):
    bn = 2048
    return pl.pallas_call(
        _proj_body,
        grid=(NP // bn,),
        in_specs=[
            pl.BlockSpec((bn, D), lambda i: (i, 0)),
            pl.BlockSpec((bn, D), lambda i: (i, 0)),
            pl.BlockSpec((D, D), lambda i: (0, 0)),
            pl.BlockSpec((D, D), lambda i: (0, 0)),
            pl.BlockSpec((D, D), lambda i: (0, 0)),
            pl.BlockSpec((D, D), lambda i: (0, 0)),
            pl.BlockSpec((1, D), lambda i: (0, 0)),
        ],
        out_specs=[
            pl.BlockSpec((D, bn), lambda i: (0, i)),
            pl.BlockSpec((D, bn), lambda i: (0, i)),
        ],
        out_shape=[jax.ShapeDtypeStruct((D, NP), jnp.float32),
                   jax.ShapeDtypeStruct((D, NP), jnp.float32)],
    )(xs, xd, w_src, w_dst, w_msg_x, w_self, b_sd)


def _qt_body(msg_ref, we_ref, qt_ref):
    qt_ref[...] = lax.dot_general(
        we_ref[...], msg_ref[...], (((0,), (1,)), ((), ())),
        preferred_element_type=jnp.float32)


def _qt_call(msg, w_e):
    be = 3200
    return pl.pallas_call(
        _qt_body,
        grid=(E_TOT // be,),
        in_specs=[
            pl.BlockSpec((be, ED), lambda i: (i, 0)),
            pl.BlockSpec((ED, D), lambda i: (0, 0)),
        ],
        out_specs=pl.BlockSpec((D, be), lambda i: (0, i)),
        out_shape=jax.ShapeDtypeStruct((D, E_TOT), jnp.float32),
    )(msg, w_e)


@functools.partial(
    pl.kernel,
    out_type=jax.ShapeDtypeStruct((D * NP,), jnp.float32),
    mesh=_mesh(),
    compiler_params=_SC_PARAMS,
    scratch_types=[
        pltpu.VMEM((CH_E,), jnp.int32),
        pltpu.VMEM((CH_E,), jnp.int32),
    ] + [pltpu.VMEM((CH_E,), jnp.float32) for _ in range(CPT)]
      + [pltpu.VMEM((NP,), jnp.float32) for _ in range(CPT)]
      + [pltpu.VMEM((NP,), jnp.float32) for _ in range(CPT)],
)
def _edge_kernel(pt_hbm, qt_hbm, src_hbm, dst_hbm, agg_out,
                 sbuf, dbuf, q0, q1, q2, q3, p0, p1, p2, p3,
                 a0, a1, a2, a3):
    qbufs = (q0, q1, q2, q3)
    pbufs = (p0, p1, p2, p3)
    abufs = (a0, a1, a2, a3)
    cid = lax.axis_index("c")
    sid = lax.axis_index("s")
    wid = cid * NS + sid
    zero16 = jnp.zeros((16,), jnp.float32)

    for r in range(CPT):
        pltpu.sync_copy(pt_hbm.at[pl.ds((CPT * wid + r) * NP, NP)], pbufs[r])

    def zero_body(i, _):
        for r in range(CPT):
            abufs[r][pl.ds(i * 16, 16)] = zero16
        return 0
    lax.fori_loop(0, NP // 16, zero_body, 0)

    def chunk_body(ci, _):
        base = ci * CH_E
        pltpu.sync_copy(src_hbm.at[pl.ds(base, CH_E)], sbuf)
        pltpu.sync_copy(dst_hbm.at[pl.ds(base, CH_E)], dbuf)
        for r in range(CPT):
            pltpu.sync_copy(
                qt_hbm.at[pl.ds((CPT * wid + r) * E_TOT + base, CH_E)],
                qbufs[r])

        def batch_body(k, _):
            sidx = sbuf[pl.ds(k * 16, 16)]
            didx = dbuf[pl.ds(k * 16, 16)]
            for r in range(CPT):
                pv = plsc.load_gather(pbufs[r], [sidx])
                qv = qbufs[r][pl.ds(k * 16, 16)]
                m = jnp.maximum(pv + qv, 0.0)
                plsc.addupdate_scatter(abufs[r], [didx], m)
            return 0
        lax.fori_loop(0, CH_E // 16, batch_body, 0)
        return 0
    lax.fori_loop(0, E_TOT // CH_E, chunk_body, 0)

    for r in range(CPT):
        pltpu.sync_copy(abufs[r],
                        agg_out.at[pl.ds((CPT * wid + r) * NP, NP)])


def _hfinal_body(aggt_ref, st_ref, benc_ref, eye_ref, h_ref):
    x = aggt_ref[...] + st_ref[...] + benc_ref[...]
    xt = lax.dot_general(x, eye_ref[...], (((0,), (0,)), ((), ())),
                         preferred_element_type=jnp.float32)
    h_ref[...] = jnp.maximum(xt, 0.0)


def _hfinal_call(aggt, st, b_enc2, eye):
    bn = 2048
    return pl.pallas_call(
        _hfinal_body,
        grid=(NP // bn,),
        in_specs=[
            pl.BlockSpec((D, bn), lambda i: (0, i)),
            pl.BlockSpec((D, bn), lambda i: (0, i)),
            pl.BlockSpec((D, 1), lambda i: (0, 0)),
            pl.BlockSpec((D, D), lambda i: (0, 0)),
        ],
        out_specs=pl.BlockSpec((bn, D), lambda i: (i, 0)),
        out_shape=jax.ShapeDtypeStruct((NP, D), jnp.float32),
    )(aggt, st, b_enc2, eye)


@functools.partial(
    pl.kernel,
    out_type=[jax.ShapeDtypeStruct((E_TOT, D), jnp.float32),
              jax.ShapeDtypeStruct((E_TOT, D), jnp.float32)],
    mesh=_mesh(),
    compiler_params=_SC_PARAMS,
    scratch_types=[
        pltpu.VMEM((CH_G,), jnp.int32),
        pltpu.VMEM((CH_G,), jnp.int32),
        pltpu.VMEM((CH_G, D), jnp.float32),
        pltpu.VMEM((CH_G, D), jnp.float32),
        pltpu.SemaphoreType.DMA,
        pltpu.SemaphoreType.DMA,
    ],
)
def _output_gather_kernel(h_hbm, src_hbm, dst_hbm, out_s, out_d,
                          sidx, didx, srows, drows, sem_s, sem_d):
    cid = lax.axis_index("c")
    sid = lax.axis_index("s")
    wid = cid * NS + sid

    def chunk_body(j, _):
        base = wid * EPT + j * CH_G
        pltpu.sync_copy(src_hbm.at[pl.ds(base, CH_G)], sidx)
        pltpu.sync_copy(dst_hbm.at[pl.ds(base, CH_G)], didx)
        cs = pltpu.async_copy(h_hbm.at[sidx], srows, sem_s)
        cd = pltpu.async_copy(h_hbm.at[didx], drows, sem_d)
        cs.wait()
        pltpu.sync_copy(srows, out_s.at[pl.ds(base, CH_G)])
        cd.wait()
        pltpu.sync_copy(drows, out_d.at[pl.ds(base, CH_G)])
        return 0
    lax.fori_loop(0, EPT // CH_G, chunk_body, 0)


def kernel(edge_index, t, msg, x_src, x_dst, W_src, b_src, W_dst, b_dst,
           W_self, W_msg, b_enc):
    del t
    src = edge_index[0].astype(jnp.int32)
    dst = edge_index[1].astype(jnp.int32)

    wmerged = _winners_kernel(src, dst)
    xs, xd = _gather_nodes_kernel(wmerged, x_src, x_dst)

    b_sd = (b_src + b_dst).reshape(1, D).astype(jnp.float32)
    pt, st = _proj_call(xs, xd, W_src, W_dst, W_msg[:D], W_self, b_sd)
    qt = _qt_call(msg, W_msg[D:])

    aggt_flat = _edge_kernel(pt.reshape(D * NP), qt.reshape(D * E_TOT),
                             src, dst)

    b_enc2 = b_enc.reshape(D, 1).astype(jnp.float32)
    eye = jnp.eye(D, dtype=jnp.float32)
    h = _hfinal_call(aggt_flat.reshape(D, NP), st, b_enc2, eye)

    h_src, h_dst = _output_gather_kernel(h, src, dst)
    return (h_src, h_dst)


# speedup vs baseline: 14.3723x; 1.9699x over previous
import functools
import jax
import jax.numpy as jnp
from jax import lax
from jax.experimental import pallas as pl
from jax.experimental.pallas import tpu as pltpu
from jax.experimental.pallas import tpu_sc as plsc

N_NODES = 10000
NP = 10240
E_TOT = 320000
D = 128
ED = 16
NC, NS = 2, 16
NW = NC * NS
EPT = E_TOT // NW
NPT = NP // NW
NPS = NP // NS
CPT = D // NW

CH_A = 2000
CH_E = 2000
CH_G = 400

_mesh = functools.partial(
    plsc.VectorSubcoreMesh, core_axis_name="c", subcore_axis_name="s")

_SC_PARAMS = pltpu.CompilerParams(needs_layout_passes=False)

@functools.partial(
    pl.kernel,
    out_type=jax.ShapeDtypeStruct((2 * NC * NP,), jnp.int32),
    mesh=_mesh(),
    compiler_params=_SC_PARAMS,
    scratch_types=[
        pltpu.VMEM((CH_A,), jnp.int32),
        pltpu.VMEM((CH_A,), jnp.int32),
        pltpu.VMEM((NP,), jnp.int32),
        pltpu.VMEM((NP,), jnp.int32),
        pltpu.VMEM((NPS,), jnp.int32),
        pltpu.VMEM((NPS,), jnp.int32),
        pltpu.VMEM_SHARED((NS * NP,), jnp.int32),
        pltpu.VMEM_SHARED((NS * NP,), jnp.int32),
    ],
)
def _winners_kernel(src_hbm, dst_hbm, wout, sbuf, dbuf, wsrc, wdst,
                    acc_s, acc_d, sh_src, sh_dst):
    cid = lax.axis_index("c")
    sid = lax.axis_index("s")
    wid = cid * NS + sid
    neg1 = jnp.full((16,), -1, jnp.int32)
    lane = lax.iota(jnp.int32, 16)
    lane_masks = [lane == i for i in range(16)]

    def init_body(i, _):
        wsrc[pl.ds(i * 16, 16)] = neg1
        wdst[pl.ds(i * 16, 16)] = neg1
        return 0
    lax.fori_loop(0, NP // 16, init_body, 0)

    for c in range(EPT // CH_A):
        base = wid * EPT + c * CH_A
        pltpu.sync_copy(src_hbm.at[pl.ds(base, CH_A)], sbuf)
        pltpu.sync_copy(dst_hbm.at[pl.ds(base, CH_A)], dbuf)

        def chunk_body(k, _):
            e0 = base + k * 16
            for buf, tbl in ((sbuf, wsrc), (dbuf, wdst)):
                idxv = buf[pl.ds(k * 16, 16)]
                e_val = e0 + lane
                for i in range(16):
                    plsc.store_scatter(tbl, [idxv], e_val, mask=lane_masks[i])
            return 0
        lax.fori_loop(0, CH_A // 16, chunk_body, 0)

    pltpu.sync_copy(wsrc, sh_src.at[pl.ds(sid * NP, NP)])
    pltpu.sync_copy(wdst, sh_dst.at[pl.ds(sid * NP, NP)])
    plsc.subcore_barrier()

    def init2(i, _):
        acc_s[pl.ds(i * 16, 16)] = neg1
        acc_d[pl.ds(i * 16, 16)] = neg1
        return 0
    lax.fori_loop(0, NPS // 16, init2, 0)

    for j in range(NS):
        pltpu.sync_copy(sh_src.at[pl.ds(j * NP, NP)], wsrc)
        pltpu.sync_copy(sh_dst.at[pl.ds(j * NP, NP)], wdst)

        def merge_body(k, _):
            off = sid * NPS + k * 16
            acc_s[pl.ds(k * 16, 16)] = jnp.maximum(
                acc_s[pl.ds(k * 16, 16)], wsrc[pl.ds(off, 16)])
            acc_d[pl.ds(k * 16, 16)] = jnp.maximum(
                acc_d[pl.ds(k * 16, 16)], wdst[pl.ds(off, 16)])
            return 0
        lax.fori_loop(0, NPS // 16, merge_body, 0)

    pltpu.sync_copy(acc_s, wout.at[pl.ds((cid * 2 + 0) * NP + sid * NPS, NPS)])
    pltpu.sync_copy(acc_d, wout.at[pl.ds((cid * 2 + 1) * NP + sid * NPS, NPS)])


@functools.partial(
    pl.kernel,
    out_type=[jax.ShapeDtypeStruct((NP, D), jnp.float32),
              jax.ShapeDtypeStruct((NP, D), jnp.float32)],
    mesh=_mesh(),
    compiler_params=_SC_PARAMS,
    scratch_types=[
        pltpu.VMEM((NPT,), jnp.int32),
        pltpu.VMEM((NPT,), jnp.int32),
        pltpu.VMEM((NPT,), jnp.int32),
        pltpu.VMEM((NPT,), jnp.int32),
        pltpu.VMEM((NPT, D), jnp.float32),
        pltpu.SemaphoreType.DMA,
    ],
)
def _gather_nodes_kernel(wm_hbm, xsrc_hbm, xdst_hbm, xs_out, xd_out,
                         w0, w1, wm, gidx, rows, sem):
    cid = lax.axis_index("c")
    sid = lax.axis_index("s")
    wid = cid * NS + sid
    nb = wid * NPT
    lane = lax.iota(jnp.int32, 16)
    zero16 = jnp.zeros((16,), jnp.float32)

    for tsel, xtab, outref in ((0, xsrc_hbm, xs_out), (1, xdst_hbm, xd_out)):
        pltpu.sync_copy(wm_hbm.at[pl.ds(tsel * NP + nb, NPT)], w0)
        pltpu.sync_copy(wm_hbm.at[pl.ds((2 + tsel) * NP + nb, NPT)], w1)

        def merge_body(k, _):
            m = jnp.maximum(w0[pl.ds(k * 16, 16)], w1[pl.ds(k * 16, 16)])
            wm[pl.ds(k * 16, 16)] = m
            gidx[pl.ds(k * 16, 16)] = jnp.maximum(m, 0)
            return 0
        lax.fori_loop(0, NPT // 16, merge_body, 0)

        pltpu.async_copy(xtab.at[gidx], rows, sem).wait()

        def mask_body(k, _):
            mv = wm[pl.ds(k * 16, 16)] < 0
            rid = k * 16 + lane
            for col in range(D):
                plsc.store_scatter(
                    rows, [rid, jnp.full((16,), col, jnp.int32)],
                    zero16, mask=mv)
            return 0
        lax.fori_loop(0, NPT // 16, mask_body, 0)

        pltpu.sync_copy(rows, outref.at[pl.ds(nb, NPT)])


def _proj_body(xs_ref, xd_ref, ws_ref, wd_ref, wmx_ref, wself_ref, bsd_ref,
               pt_ref, st_ref):
    xproj = lax.dot_general(xs_ref[...], ws_ref[...], (((1,), (1,)), ((), ())),
                            preferred_element_type=jnp.float32)
    xproj += lax.dot_general(xd_ref[...], wd_ref[...], (((1,), (1,)), ((), ())),
                             preferred_element_type=jnp.float32)
    xproj += bsd_ref[...]
    pt_ref[...] = lax.dot_general(wmx_ref[...], xproj, (((0,), (1,)), ((), ())),
                                  preferred_element_type=jnp.float32)
    st_ref[...] = lax.dot_general(wself_ref[...], xproj, (((0,), (1,)), ((), ())),
                                  preferred_element_type=jnp.float32)


def _proj_call(xs, xd, w_src, w_dst, w_msg_x, w_self, b_sd):
    bn = 2048
    return pl.pallas_call(
        _proj_body,
        grid=(NP // bn,),
        in_specs=[
            pl.BlockSpec((bn, D), lambda i: (i, 0)),
            pl.BlockSpec((bn, D), lambda i: (i, 0)),
            pl.BlockSpec((D, D), lambda i: (0, 0)),
            pl.BlockSpec((D, D), lambda i: (0, 0)),
            pl.BlockSpec((D, D), lambda i: (0, 0)),
            pl.BlockSpec((D, D), lambda i: (0, 0)),
            pl.BlockSpec((1, D), lambda i: (0, 0)),
        ],
        out_specs=[
            pl.BlockSpec((D, bn), lambda i: (0, i)),
            pl.BlockSpec((D, bn), lambda i: (0, i)),
        ],
        out_shape=[jax.ShapeDtypeStruct((D, NP), jnp.float32),
                   jax.ShapeDtypeStruct((D, NP), jnp.float32)],
    )(xs, xd, w_src, w_dst, w_msg_x, w_self, b_sd)


def _qt_body(msg_ref, we_ref, qt_ref):
    qt_ref[...] = lax.dot_general(
        we_ref[...], msg_ref[...], (((0,), (1,)), ((), ())),
        preferred_element_type=jnp.float32)


def _qt_call(msg, w_e):
    be = 3200
    return pl.pallas_call(
        _qt_body,
        grid=(E_TOT // be,),
        in_specs=[
            pl.BlockSpec((be, ED), lambda i: (i, 0)),
            pl.BlockSpec((ED, D), lambda i: (0, 0)),
        ],
        out_specs=pl.BlockSpec((D, be), lambda i: (0, i)),
        out_shape=jax.ShapeDtypeStruct((D, E_TOT), jnp.float32),
    )(msg, w_e)


@functools.partial(
    pl.kernel,
    out_type=jax.ShapeDtypeStruct((D * NP,), jnp.float32),
    mesh=_mesh(),
    compiler_params=_SC_PARAMS,
    scratch_types=(
        [pltpu.VMEM((CH_E,), jnp.int32) for _ in range(4)]
        + [pltpu.VMEM((CH_E,), jnp.float32) for _ in range(2 * CPT)]
        + [pltpu.VMEM((NP,), jnp.float32) for _ in range(CPT)]
        + [pltpu.VMEM((NP,), jnp.float32) for _ in range(CPT)]
        + [pltpu.SemaphoreType.DMA, pltpu.SemaphoreType.DMA]
    ),
)
def _edge_kernel(pt_hbm, qt_hbm, src_hbm, dst_hbm, agg_out,
                 s0, d0, s1, d1, qa0, qa1, qa2, qa3, qb0, qb1, qb2, qb3,
                 p0, p1, p2, p3, a0, a1, a2, a3, semA, semB):
    bufs = (
        (s0, d0, (qa0, qa1, qa2, qa3), semA),
        (s1, d1, (qb0, qb1, qb2, qb3), semB),
    )
    pbufs = (p0, p1, p2, p3)
    abufs = (a0, a1, a2, a3)
    cid = lax.axis_index("c")
    sid = lax.axis_index("s")
    wid = cid * NS + sid
    zero16 = jnp.zeros((16,), jnp.float32)
    nchunks = E_TOT // CH_E

    def issue(base, which):
        sb, db, qb, sem = bufs[which]
        pltpu.async_copy(src_hbm.at[pl.ds(base, CH_E)], sb, sem)
        pltpu.async_copy(dst_hbm.at[pl.ds(base, CH_E)], db, sem)
        for r in range(CPT):
            pltpu.async_copy(
                qt_hbm.at[pl.ds((CPT * wid + r) * E_TOT + base, CH_E)],
                qb[r], sem)

    def drain(which):
        sb, db, qb, sem = bufs[which]
        pltpu.make_async_copy(src_hbm.at[pl.ds(0, CH_E)], sb, sem).wait()
        pltpu.make_async_copy(src_hbm.at[pl.ds(0, CH_E)], db, sem).wait()
        for r in range(CPT):
            pltpu.make_async_copy(qt_hbm.at[pl.ds(0, CH_E)], qb[r], sem).wait()

    def compute(which):
        sb, db, qb, _ = bufs[which]

        @plsc.parallel_loop(0, CH_E // 16, unroll=8)
        def batch_body(k):
            sidx = sb[pl.ds(k * 16, 16)]
            didx = db[pl.ds(k * 16, 16)]
            for r in range(CPT):
                pv = plsc.load_gather(pbufs[r], [sidx])
                qv = qb[r][pl.ds(k * 16, 16)]
                m = jnp.maximum(pv + qv, 0.0)
                plsc.addupdate_scatter(abufs[r], [didx], m)

    for r in range(CPT):
        pltpu.sync_copy(pt_hbm.at[pl.ds((CPT * wid + r) * NP, NP)], pbufs[r])
    issue(0, 0)

    def zero_body(i, _):
        for r in range(CPT):
            abufs[r][pl.ds(i * 16, 16)] = zero16
        return 0
    lax.fori_loop(0, NP // 16, zero_body, 0)

    def pair_body(mi, _):
        c0 = 2 * mi * CH_E
        drain(0)
        issue(c0 + CH_E, 1)
        compute(0)
        drain(1)

        @pl.when(2 * mi + 2 < nchunks)
        def _():
            issue(c0 + 2 * CH_E, 0)
        compute(1)
        return 0
    lax.fori_loop(0, nchunks // 2, pair_body, 0)

    for r in range(CPT):
        pltpu.sync_copy(abufs[r],
                        agg_out.at[pl.ds((CPT * wid + r) * NP, NP)])


def _hfinal_body(aggt_ref, st_ref, benc_ref, eye_ref, h_ref):
    x = aggt_ref[...] + st_ref[...] + benc_ref[...]
    xt = lax.dot_general(x, eye_ref[...], (((0,), (0,)), ((), ())),
                         preferred_element_type=jnp.float32)
    h_ref[...] = jnp.maximum(xt, 0.0)


def _hfinal_call(aggt, st, b_enc2, eye):
    bn = 2048
    return pl.pallas_call(
        _hfinal_body,
        grid=(NP // bn,),
        in_specs=[
            pl.BlockSpec((D, bn), lambda i: (0, i)),
            pl.BlockSpec((D, bn), lambda i: (0, i)),
            pl.BlockSpec((D, 1), lambda i: (0, 0)),
            pl.BlockSpec((D, D), lambda i: (0, 0)),
        ],
        out_specs=pl.BlockSpec((bn, D), lambda i: (i, 0)),
        out_shape=jax.ShapeDtypeStruct((NP, D), jnp.float32),
    )(aggt, st, b_enc2, eye)


@functools.partial(
    pl.kernel,
    out_type=[jax.ShapeDtypeStruct((E_TOT, D), jnp.float32),
              jax.ShapeDtypeStruct((E_TOT, D), jnp.float32)],
    mesh=_mesh(),
    compiler_params=_SC_PARAMS,
    scratch_types=[
        pltpu.VMEM((CH_G,), jnp.int32),
        pltpu.VMEM((CH_G,), jnp.int32),
        pltpu.VMEM((CH_G, D), jnp.float32),
        pltpu.VMEM((CH_G, D), jnp.float32),
        pltpu.SemaphoreType.DMA,
        pltpu.SemaphoreType.DMA,
    ],
)
def _output_gather_kernel(h_hbm, src_hbm, dst_hbm, out_s, out_d,
                          sidx, didx, srows, drows, sem_s, sem_d):
    cid = lax.axis_index("c")
    sid = lax.axis_index("s")
    wid = cid * NS + sid

    def chunk_body(j, _):
        base = wid * EPT + j * CH_G
        pltpu.sync_copy(src_hbm.at[pl.ds(base, CH_G)], sidx)
        pltpu.sync_copy(dst_hbm.at[pl.ds(base, CH_G)], didx)
        cs = pltpu.async_copy(h_hbm.at[sidx], srows, sem_s)
        cd = pltpu.async_copy(h_hbm.at[didx], drows, sem_d)
        cs.wait()
        pltpu.sync_copy(srows, out_s.at[pl.ds(base, CH_G)])
        cd.wait()
        pltpu.sync_copy(drows, out_d.at[pl.ds(base, CH_G)])
        return 0
    lax.fori_loop(0, EPT // CH_G, chunk_body, 0)


def kernel(edge_index, t, msg, x_src, x_dst, W_src, b_src, W_dst, b_dst,
           W_self, W_msg, b_enc):
    del t
    src = edge_index[0].astype(jnp.int32)
    dst = edge_index[1].astype(jnp.int32)

    wmerged = _winners_kernel(src, dst)
    xs, xd = _gather_nodes_kernel(wmerged, x_src, x_dst)

    b_sd = (b_src + b_dst).reshape(1, D).astype(jnp.float32)
    pt, st = _proj_call(xs, xd, W_src, W_dst, W_msg[:D], W_self, b_sd)
    qt = _qt_call(msg, W_msg[D:])

    aggt_flat = _edge_kernel(pt.reshape(D * NP), qt.reshape(D * E_TOT),
                             src, dst)

    b_enc2 = b_enc.reshape(D, 1).astype(jnp.float32)
    eye = jnp.eye(D, dtype=jnp.float32)
    h = _hfinal_call(aggt_flat.reshape(D, NP), st, b_enc2, eye)

    h_src, h_dst = _output_gather_kernel(h, src, dst)
    return (h_src, h_dst)
